# baseline (speedup 1.0000x reference)
import jax
import jax.numpy as jnp
from jax.experimental import pallas as pl
from jax.experimental.pallas import tpu as pltpu

B = 8
H = 1024
S = 64
EPS = 1e-12


def _fused_kernel(x_hbm, pos_hbm, tok_hbm, gamma_ref, beta_ref, out_hbm,
                  xbuf, obuf, posbuf, tokbuf, insems, outsems, csem):
    for i in range(B):
        pltpu.make_async_copy(x_hbm.at[i], xbuf.at[i], insems.at[i]).start()
    pltpu.make_async_copy(pos_hbm, posbuf, csem).start()
    pltpu.make_async_copy(tok_hbm.at[pl.ds(1, 1)], tokbuf, csem).start()
    pltpu.make_async_copy(pos_hbm, posbuf, csem).wait()
    pltpu.make_async_copy(tok_hbm.at[pl.ds(1, 1)], tokbuf, csem).wait()

    bias = posbuf[...] + tokbuf[...]
    bias2 = jnp.concatenate([bias, bias], axis=0)
    gamma = gamma_ref[...]
    beta = beta_ref[...]

    for c in range(B // 2):
        pltpu.make_async_copy(x_hbm.at[2 * c], xbuf.at[2 * c], insems.at[2 * c]).wait()
        pltpu.make_async_copy(x_hbm.at[2 * c + 1], xbuf.at[2 * c + 1], insems.at[2 * c + 1]).wait()
        e = xbuf[2 * c:2 * c + 2].reshape(2 * S, H).astype(jnp.float32) + bias2
        m1 = jnp.sum(e, axis=1, keepdims=True) * (1.0 / H)
        m2 = jnp.sum(e * e, axis=1, keepdims=True) * (1.0 / H)
        var = m2 - m1 * m1
        inv = jax.lax.rsqrt(var + EPS)
        obuf[2 * c:2 * c + 2] = ((e - m1) * inv * gamma + beta).reshape(2, S, H)
        pltpu.make_async_copy(obuf.at[2 * c], out_hbm.at[2 * c], outsems.at[2 * c]).start()
        pltpu.make_async_copy(obuf.at[2 * c + 1], out_hbm.at[2 * c + 1], outsems.at[2 * c + 1]).start()
    for i in range(B):
        pltpu.make_async_copy(obuf.at[i], out_hbm.at[i], outsems.at[i]).wait()


def kernel(input_ids, pos_table, tok_table, ln_gamma, ln_beta):
    xt = jnp.transpose(input_ids, (0, 2, 1)).astype(jnp.bfloat16)
    gamma2 = ln_gamma.reshape(1, H)
    beta2 = ln_beta.reshape(1, H)
    out = pl.pallas_call(
        _fused_kernel,
        in_specs=[
            pl.BlockSpec(memory_space=pl.ANY),
            pl.BlockSpec(memory_space=pl.ANY),
            pl.BlockSpec(memory_space=pl.ANY),
            pl.BlockSpec(memory_space=pltpu.MemorySpace.VMEM),
            pl.BlockSpec(memory_space=pltpu.MemorySpace.VMEM),
        ],
        out_specs=pl.BlockSpec(memory_space=pl.ANY),
        out_shape=jax.ShapeDtypeStruct((B, S, H), jnp.float32),
        scratch_shapes=[
            pltpu.VMEM((B, S, H), jnp.bfloat16),
            pltpu.VMEM((B, S, H), jnp.float32),
            pltpu.VMEM((S, H), jnp.float32),
            pltpu.VMEM((1, H), jnp.float32),
            pltpu.SemaphoreType.DMA((B,)),
            pltpu.SemaphoreType.DMA((B,)),
            pltpu.SemaphoreType.DMA,
        ],
    )(xt, pos_table, tok_table, gamma2, beta2)
    return out


# speedup vs baseline: 1.4631x; 1.4631x over previous
import jax
import jax.numpy as jnp
from jax.experimental import pallas as pl
from jax.experimental.pallas import tpu as pltpu

B = 8
H = 1024
S = 64
EPS = 1e-12
CB = 2
NC = B // CB


def _fused_kernel(x_hbm, pos_hbm, tok_hbm, gamma_ref, beta_ref, out_hbm,
                  xbuf, obuf, posbuf, tokbuf, insems, outsems, csem):
    def start_in(c):
        for j in range(CB):
            i = CB * c + j
            pltpu.make_async_copy(x_hbm.at[i], xbuf.at[i], insems.at[i]).start()

    start_in(0)
    pltpu.make_async_copy(pos_hbm, posbuf, csem).start()
    pltpu.make_async_copy(tok_hbm.at[pl.ds(1, 1)], tokbuf, csem).start()
    start_in(1)
    pltpu.make_async_copy(pos_hbm, posbuf, csem).wait()
    pltpu.make_async_copy(tok_hbm.at[pl.ds(1, 1)], tokbuf, csem).wait()

    bias = posbuf[...] + tokbuf[...]
    bias2 = jnp.concatenate([bias] * CB, axis=0)
    gamma = gamma_ref[...]
    beta = beta_ref[...]

    for c in range(NC):
        if c + 2 < NC:
            start_in(c + 2)
        for j in range(CB):
            i = CB * c + j
            pltpu.make_async_copy(x_hbm.at[i], xbuf.at[i], insems.at[i]).wait()
        e = xbuf[CB * c:CB * (c + 1)].reshape(CB * S, H) + bias2
        m1 = jnp.sum(e, axis=1, keepdims=True) * (1.0 / H)
        m2 = jnp.sum(e * e, axis=1, keepdims=True) * (1.0 / H)
        var = m2 - m1 * m1
        inv = jax.lax.rsqrt(var + EPS)
        obuf[CB * c:CB * (c + 1)] = ((e - m1) * inv * gamma + beta).reshape(CB, S, H)
        for j in range(CB):
            i = CB * c + j
            pltpu.make_async_copy(obuf.at[i], out_hbm.at[i], outsems.at[i]).start()
    for i in range(B):
        pltpu.make_async_copy(obuf.at[i], out_hbm.at[i], outsems.at[i]).wait()


def kernel(input_ids, pos_table, tok_table, ln_gamma, ln_beta):
    xt = jnp.transpose(input_ids, (0, 2, 1))
    gamma2 = ln_gamma.reshape(1, H)
    beta2 = ln_beta.reshape(1, H)
    out = pl.pallas_call(
        _fused_kernel,
        in_specs=[
            pl.BlockSpec(memory_space=pl.ANY),
            pl.BlockSpec(memory_space=pl.ANY),
            pl.BlockSpec(memory_space=pl.ANY),
            pl.BlockSpec(memory_space=pltpu.MemorySpace.VMEM),
            pl.BlockSpec(memory_space=pltpu.MemorySpace.VMEM),
        ],
        out_specs=pl.BlockSpec(memory_space=pl.ANY),
        out_shape=jax.ShapeDtypeStruct((B, S, H), jnp.float32),
        scratch_shapes=[
            pltpu.VMEM((B, S, H), jnp.float32),
            pltpu.VMEM((B, S, H), jnp.float32),
            pltpu.VMEM((S, H), jnp.float32),
            pltpu.VMEM((1, H), jnp.float32),
            pltpu.SemaphoreType.DMA((B,)),
            pltpu.SemaphoreType.DMA((B,)),
            pltpu.SemaphoreType.DMA,
        ],
    )(xt, pos_table, tok_table, gamma2, beta2)
    return out


# speedup vs baseline: 1.5683x; 1.0719x over previous
import jax
import jax.numpy as jnp
from jax.experimental import pallas as pl
from jax.experimental.pallas import tpu as pltpu

B = 8
H = 1024
S = 64
EPS = 1e-12


def _fused_kernel(x_hbm, pos_hbm, tok_hbm, gamma_ref, beta_ref, out_hbm,
                  xbuf, obuf, posbuf, tokbuf, insems, outsems, csem):
    for i in range(B):
        pltpu.make_async_copy(x_hbm.at[i], xbuf.at[i], insems.at[i]).start()
    pltpu.make_async_copy(pos_hbm, posbuf, csem).start()
    pltpu.make_async_copy(tok_hbm.at[pl.ds(1, 1)], tokbuf, csem).start()
    pltpu.make_async_copy(pos_hbm, posbuf, csem).wait()
    pltpu.make_async_copy(tok_hbm.at[pl.ds(1, 1)], tokbuf, csem).wait()

    bias = posbuf[...] + tokbuf[...]
    bias2 = jnp.concatenate([bias, bias], axis=0)
    gamma = gamma_ref[...]
    beta = beta_ref[...]

    for c in range(B // 2):
        pltpu.make_async_copy(x_hbm.at[2 * c], xbuf.at[2 * c], insems.at[2 * c]).wait()
        pltpu.make_async_copy(x_hbm.at[2 * c + 1], xbuf.at[2 * c + 1], insems.at[2 * c + 1]).wait()
        e = xbuf[2 * c:2 * c + 2].reshape(2 * S, H) + bias2
        m1 = jnp.sum(e, axis=1, keepdims=True) * (1.0 / H)
        m2 = jnp.sum(e * e, axis=1, keepdims=True) * (1.0 / H)
        var = m2 - m1 * m1
        inv = jax.lax.rsqrt(var + EPS)
        obuf[2 * c:2 * c + 2] = ((e - m1) * inv * gamma + beta).reshape(2, S, H)
        pltpu.make_async_copy(obuf.at[2 * c], out_hbm.at[2 * c], outsems.at[2 * c]).start()
        pltpu.make_async_copy(obuf.at[2 * c + 1], out_hbm.at[2 * c + 1], outsems.at[2 * c + 1]).start()
    for i in range(B):
        pltpu.make_async_copy(obuf.at[i], out_hbm.at[i], outsems.at[i]).wait()


def kernel(input_ids, pos_table, tok_table, ln_gamma, ln_beta):
    xt = jnp.transpose(input_ids, (0, 2, 1))
    gamma2 = ln_gamma.reshape(1, H)
    beta2 = ln_beta.reshape(1, H)
    out = pl.pallas_call(
        _fused_kernel,
        in_specs=[
            pl.BlockSpec(memory_space=pl.ANY),
            pl.BlockSpec(memory_space=pl.ANY),
            pl.BlockSpec(memory_space=pl.ANY),
            pl.BlockSpec(memory_space=pltpu.MemorySpace.VMEM),
            pl.BlockSpec(memory_space=pltpu.MemorySpace.VMEM),
        ],
        out_specs=pl.BlockSpec(memory_space=pl.ANY),
        out_shape=jax.ShapeDtypeStruct((B, S, H), jnp.float32),
        scratch_shapes=[
            pltpu.VMEM((B, S, H), jnp.float32),
            pltpu.VMEM((B, S, H), jnp.float32),
            pltpu.VMEM((S, H), jnp.float32),
            pltpu.VMEM((1, H), jnp.float32),
            pltpu.SemaphoreType.DMA((B,)),
            pltpu.SemaphoreType.DMA((B,)),
            pltpu.SemaphoreType.DMA,
        ],
    )(xt, pos_table, tok_table, gamma2, beta2)
    return out


# speedup vs baseline: 1.6069x; 1.0246x over previous
import jax
import jax.numpy as jnp
from jax.experimental import pallas as pl
from jax.experimental.pallas import tpu as pltpu

B = 8
H = 1024
S = 64
EPS = 1e-12


def _fused_kernel(x_hbm, pos_hbm, tok_hbm, gamma_ref, beta_ref, out_hbm,
                  xbuf, obuf, posbuf, tokbuf, insems, outsems, csem):
    for i in range(B // 2):
        pltpu.make_async_copy(x_hbm.at[pl.ds(2 * i, 2)], xbuf.at[pl.ds(2 * i, 2)], insems.at[i]).start()
    pltpu.make_async_copy(pos_hbm, posbuf, csem).start()
    pltpu.make_async_copy(tok_hbm.at[pl.ds(1, 1)], tokbuf, csem).start()
    pltpu.make_async_copy(pos_hbm, posbuf, csem).wait()
    pltpu.make_async_copy(tok_hbm.at[pl.ds(1, 1)], tokbuf, csem).wait()

    bias = posbuf[...] + tokbuf[...]
    bias2 = jnp.concatenate([bias, bias], axis=0)
    gamma = gamma_ref[...]
    beta = beta_ref[...]

    for c in range(B // 2):
        pltpu.make_async_copy(x_hbm.at[pl.ds(2 * c, 2)], xbuf.at[pl.ds(2 * c, 2)], insems.at[c]).wait()
        e = xbuf[2 * c:2 * c + 2].reshape(2 * S, H) + bias2
        m1 = jnp.sum(e, axis=1, keepdims=True) * (1.0 / H)
        m2 = jnp.sum(e * e, axis=1, keepdims=True) * (1.0 / H)
        var = m2 - m1 * m1
        inv = jax.lax.rsqrt(var + EPS)
        obuf[2 * c:2 * c + 2] = ((e - m1) * inv * gamma + beta).reshape(2, S, H)
        pltpu.make_async_copy(obuf.at[pl.ds(2 * c, 2)], out_hbm.at[pl.ds(2 * c, 2)], outsems.at[c]).start()
    for c in range(B // 2):
        pltpu.make_async_copy(obuf.at[pl.ds(2 * c, 2)], out_hbm.at[pl.ds(2 * c, 2)], outsems.at[c]).wait()


def kernel(input_ids, pos_table, tok_table, ln_gamma, ln_beta):
    xt = jnp.transpose(input_ids, (0, 2, 1))
    gamma2 = ln_gamma.reshape(1, H)
    beta2 = ln_beta.reshape(1, H)
    out = pl.pallas_call(
        _fused_kernel,
        in_specs=[
            pl.BlockSpec(memory_space=pl.ANY),
            pl.BlockSpec(memory_space=pl.ANY),
            pl.BlockSpec(memory_space=pl.ANY),
            pl.BlockSpec(memory_space=pltpu.MemorySpace.VMEM),
            pl.BlockSpec(memory_space=pltpu.MemorySpace.VMEM),
        ],
        out_specs=pl.BlockSpec(memory_space=pl.ANY),
        out_shape=jax.ShapeDtypeStruct((B, S, H), jnp.float32),
        scratch_shapes=[
            pltpu.VMEM((B, S, H), jnp.float32),
            pltpu.VMEM((B, S, H), jnp.float32),
            pltpu.VMEM((S, H), jnp.float32),
            pltpu.VMEM((1, H), jnp.float32),
            pltpu.SemaphoreType.DMA((B,)),
            pltpu.SemaphoreType.DMA((B,)),
            pltpu.SemaphoreType.DMA,
        ],
    )(xt, pos_table, tok_table, gamma2, beta2)
    return out
